# Initial kernel scaffold; baseline (speedup 1.0000x reference)
import jax
import jax.numpy as jnp
from jax.experimental import pallas as pl


def kernel(x, wids, lora_A):
    raise NotImplementedError("write your pallas kernel here")


# speedup vs baseline: 2.0502x; 2.0502x over previous
import functools

import jax
import jax.numpy as jnp
from jax import lax
from jax.experimental import pallas as pl
from jax.experimental.pallas import tpu as pltpu
from jax.experimental.pallas import tpu_sc as plsc

B = 512
D = 4096
R = 64
N = 64
G = 4
STEPS = N // G


def _mm_body(x_ref, a_ref, y_ref):
    w = jnp.concatenate([a_ref[pl.ds(i * D, D), :] for i in range(G)], axis=1)
    yblk = lax.dot_general(
        x_ref[...], w, (((1,), (0,)), ((), ())),
        preferred_element_type=jnp.float32)
    for i in range(G):
        s = yblk[:, i * R:(i + 1) * R]
        y_ref[:, pl.ds(i * 2 * R, R)] = s
        y_ref[:, pl.ds(i * 2 * R + R, R)] = s


def _dense_all_adapters(x2d, lora_A):
    return pl.pallas_call(
        _mm_body,
        grid=(STEPS,),
        in_specs=[
            pl.BlockSpec((B, D), lambda g: (0, 0)),
            pl.BlockSpec((G * D, R), lambda g: (g, 0)),
        ],
        out_specs=pl.BlockSpec((B, G * 2 * R), lambda g: (0, g)),
        out_shape=jax.ShapeDtypeStruct((B, N * 2 * R), jnp.float32),
    )(x2d, lora_A.reshape(N * D, R))


_NC = 2
_NS = 16
_NW = _NC * _NS
_BPW = B // _NW


@functools.cache
def _make_route_gather():
    @functools.partial(
        pl.kernel,
        out_type=jax.ShapeDtypeStruct((B, 2 * R), jnp.float32),
        mesh=plsc.VectorSubcoreMesh(core_axis_name="c", subcore_axis_name="s"),
        scratch_types=[
            pltpu.VMEM((_BPW,), jnp.int32),
            pltpu.VMEM((_BPW,), jnp.int32),
            pltpu.VMEM((_BPW, 2 * R), jnp.float32),
            pltpu.SemaphoreType.DMA,
        ],
    )
    def _route_gather(y_hbm, wids_hbm, out_hbm, wids_v, idx_v, rows_v, sem):
        wid = lax.axis_index("s") * _NC + lax.axis_index("c")
        base = wid * _BPW
        pltpu.sync_copy(wids_hbm.at[pl.ds(base, _BPW)], wids_v)
        lane = lax.iota(jnp.int32, _BPW)
        idx_v[...] = (base + lane) * N + wids_v[...]
        pltpu.async_copy(y_hbm.at[idx_v], rows_v, sem).wait()
        pltpu.sync_copy(rows_v, out_hbm.at[pl.ds(base, _BPW)])

    return _route_gather


def kernel(x, wids, lora_A):
    x2d = x.reshape(B, D).astype(jnp.bfloat16)
    lora_A = lora_A.astype(jnp.bfloat16)
    y = _dense_all_adapters(x2d, lora_A)
    h = _make_route_gather()(y.reshape(B * N, 2 * R), wids)
    return h[:, :R].astype(jnp.float16).reshape(B, 1, R)
